# baseline (speedup 1.0000x reference)
import jax
import jax.numpy as jnp
from jax.experimental import pallas as pl

TOP_K = 8
BLOCK_M = 512


def _router_kernel(x_ref, w_ref, b_ref, rw_ref, probs_ref):
    logits = jnp.dot(x_ref[...], w_ref[...],
                     preferred_element_type=jnp.float32)
    logits = logits + b_ref[...]
    m = jnp.max(logits, axis=-1, keepdims=True)
    e = jnp.exp(logits - m)
    probs = e / jnp.sum(e, axis=-1, keepdims=True)

    n = probs.shape[-1]
    idx = jax.lax.broadcasted_iota(jnp.int32, probs.shape, 1)
    work = probs
    mask = jnp.zeros(probs.shape, dtype=jnp.bool_)
    for _ in range(TOP_K):
        mx = jnp.max(work, axis=-1, keepdims=True)
        cand = jnp.where(work == mx, idx, n)
        amin = jnp.min(cand, axis=-1, keepdims=True)
        one = idx == amin
        mask = jnp.logical_or(mask, one)
        work = jnp.where(one, -jnp.inf, work)

    probs_ref[...] = probs
    rw_ref[...] = jnp.where(mask, probs, 0.0)


@jax.jit
def kernel(x, W, b):
    C = x.shape[-1]
    x_flat = x.reshape(-1, C)
    M = x_flat.shape[0]
    N = W.shape[-1]
    b2 = b.reshape(1, N)

    grid = (M // BLOCK_M,)
    rw, probs = pl.pallas_call(
        _router_kernel,
        grid=grid,
        in_specs=[
            pl.BlockSpec((BLOCK_M, C), lambda i: (i, 0)),
            pl.BlockSpec((C, N), lambda i: (0, 0)),
            pl.BlockSpec((1, N), lambda i: (0, 0)),
        ],
        out_specs=[
            pl.BlockSpec((BLOCK_M, N), lambda i: (i, 0)),
            pl.BlockSpec((BLOCK_M, N), lambda i: (i, 0)),
        ],
        out_shape=[
            jax.ShapeDtypeStruct((M, N), jnp.float32),
            jax.ShapeDtypeStruct((M, N), jnp.float32),
        ],
    )(x_flat, W, b2)
    return rw, probs


# speedup vs baseline: 1.3040x; 1.3040x over previous
import jax
import jax.numpy as jnp
from jax.experimental import pallas as pl

TOP_K = 8
BLOCK_M = 1024


def _router_kernel(x_ref, w_ref, b_ref, rw_ref, probs_ref):
    logits = jnp.dot(x_ref[...], w_ref[...],
                     preferred_element_type=jnp.float32)
    logits = logits + b_ref[...]
    m = jnp.max(logits, axis=-1, keepdims=True)
    e = jnp.exp(logits - m)
    probs = e / jnp.sum(e, axis=-1, keepdims=True)

    idx = jax.lax.broadcasted_iota(jnp.int32, probs.shape, 1)
    work = probs
    mask = jnp.zeros(probs.shape, dtype=jnp.bool_)
    for _ in range(TOP_K):
        a = jnp.argmax(work, axis=-1)
        one = idx == a[:, None]
        mask = jnp.logical_or(mask, one)
        work = jnp.where(one, -jnp.inf, work)

    probs_ref[...] = probs
    rw_ref[...] = jnp.where(mask, probs, 0.0)


@jax.jit
def kernel(x, W, b):
    C = x.shape[-1]
    x_flat = x.reshape(-1, C)
    M = x_flat.shape[0]
    N = W.shape[-1]
    b2 = b.reshape(1, N)

    grid = (M // BLOCK_M,)
    rw, probs = pl.pallas_call(
        _router_kernel,
        grid=grid,
        in_specs=[
            pl.BlockSpec((BLOCK_M, C), lambda i: (i, 0)),
            pl.BlockSpec((C, N), lambda i: (0, 0)),
            pl.BlockSpec((1, N), lambda i: (0, 0)),
        ],
        out_specs=[
            pl.BlockSpec((BLOCK_M, N), lambda i: (i, 0)),
            pl.BlockSpec((BLOCK_M, N), lambda i: (i, 0)),
        ],
        out_shape=[
            jax.ShapeDtypeStruct((M, N), jnp.float32),
            jax.ShapeDtypeStruct((M, N), jnp.float32),
        ],
    )(x_flat, W, b2)
    return rw, probs
